# Initial kernel scaffold; baseline (speedup 1.0000x reference)
import jax
import jax.numpy as jnp
from jax.experimental import pallas as pl


def kernel(x, adj, W1, W2, cluster_layer):
    raise NotImplementedError("write your pallas kernel here")


# speedup vs baseline: 1.6066x; 1.6066x over previous
import functools

import jax
import jax.numpy as jnp
from jax import lax
from jax.experimental import pallas as pl
from jax.experimental.pallas import tpu as pltpu

N = 4096
D = 512
H1 = 256
H2 = 128
C = 16
HE = H1 + H2


def _s1_body(x_ref, w1_ref, o_ref):
    o_ref[...] = jnp.dot(x_ref[...], w1_ref[...],
                         preferred_element_type=jnp.float32)


def _gc1_body(adj_ref, s1_ref, w2_ref, z1_ref, s2_ref):
    z1 = jnp.maximum(
        jnp.dot(adj_ref[...], s1_ref[...], preferred_element_type=jnp.float32),
        0.0)
    z1_ref[...] = z1
    s2_ref[...] = jnp.dot(z1, w2_ref[...], preferred_element_type=jnp.float32)


def _gc2_body(adj_ref, s2_ref, z1_ref, clt_ref, enc_ref, q_ref):
    z2 = jnp.dot(adj_ref[...], s2_ref[...], preferred_element_type=jnp.float32)
    enc = jnp.concatenate([z1_ref[...], z2], axis=1)
    enc_ref[...] = enc
    clt = clt_ref[...]
    en2 = jnp.sum(enc * enc, axis=1, keepdims=True)
    cn2 = jnp.sum(clt * clt, axis=0, keepdims=True)
    cross = jnp.dot(enc, clt, preferred_element_type=jnp.float32)
    dist = en2 - 2.0 * cross + cn2
    q = 1.0 / (1.0 + dist)
    q_ref[...] = q / jnp.sum(q, axis=1, keepdims=True)


def _dec_body(encb_ref, enc_ref, o_ref):
    s = lax.dot_general(encb_ref[...], enc_ref[...],
                        (((1,), (1,)), ((), ())),
                        preferred_element_type=jnp.float32)
    o_ref[...] = jax.nn.sigmoid(s)


@jax.jit
def kernel(x, adj, W1, W2, cluster_layer):
    bma = 512
    s1 = pl.pallas_call(
        _s1_body,
        grid=(N // bma,),
        in_specs=[
            pl.BlockSpec((bma, D), lambda i: (i, 0)),
            pl.BlockSpec((D, H1), lambda i: (0, 0)),
        ],
        out_specs=pl.BlockSpec((bma, H1), lambda i: (i, 0)),
        out_shape=jax.ShapeDtypeStruct((N, H1), jnp.float32),
    )(x, W1)

    bm = 256
    z1, s2 = pl.pallas_call(
        _gc1_body,
        grid=(N // bm,),
        in_specs=[
            pl.BlockSpec((bm, N), lambda i: (i, 0)),
            pl.BlockSpec((N, H1), lambda i: (0, 0)),
            pl.BlockSpec((H1, H2), lambda i: (0, 0)),
        ],
        out_specs=[
            pl.BlockSpec((bm, H1), lambda i: (i, 0)),
            pl.BlockSpec((bm, H2), lambda i: (i, 0)),
        ],
        out_shape=[
            jax.ShapeDtypeStruct((N, H1), jnp.float32),
            jax.ShapeDtypeStruct((N, H2), jnp.float32),
        ],
    )(adj, s1, W2)

    enc, q = pl.pallas_call(
        _gc2_body,
        grid=(N // bm,),
        in_specs=[
            pl.BlockSpec((bm, N), lambda i: (i, 0)),
            pl.BlockSpec((N, H2), lambda i: (0, 0)),
            pl.BlockSpec((bm, H1), lambda i: (i, 0)),
            pl.BlockSpec((HE, C), lambda i: (0, 0)),
        ],
        out_specs=[
            pl.BlockSpec((bm, HE), lambda i: (i, 0)),
            pl.BlockSpec((bm, C), lambda i: (i, 0)),
        ],
        out_shape=[
            jax.ShapeDtypeStruct((N, HE), jnp.float32),
            jax.ShapeDtypeStruct((N, C), jnp.float32),
        ],
    )(adj, s2, z1, cluster_layer.T)

    dec = pl.pallas_call(
        _dec_body,
        grid=(N // bm,),
        in_specs=[
            pl.BlockSpec((bm, HE), lambda i: (i, 0)),
            pl.BlockSpec((N, HE), lambda i: (0, 0)),
        ],
        out_specs=pl.BlockSpec((bm, N), lambda i: (i, 0)),
        out_shape=jax.ShapeDtypeStruct((N, N), jnp.float32),
    )(enc, enc)

    return (enc, dec, q)


# speedup vs baseline: 1.6211x; 1.0090x over previous
import functools

import jax
import jax.numpy as jnp
from jax import lax
from jax.experimental import pallas as pl
from jax.experimental.pallas import tpu as pltpu

N = 4096
D = 512
H1 = 256
H2 = 128
C = 16
HE = H1 + H2


def _bf(a):
    return a.astype(jnp.bfloat16)


def _s1_body(x_ref, w1_ref, o_ref):
    o_ref[...] = jnp.dot(_bf(x_ref[...]), _bf(w1_ref[...]),
                         preferred_element_type=jnp.float32)


def _gc1_body(adj_ref, s1_ref, w2_ref, z1_ref, s2_ref):
    z1 = jnp.maximum(
        jnp.dot(_bf(adj_ref[...]), _bf(s1_ref[...]),
                preferred_element_type=jnp.float32),
        0.0)
    z1_ref[...] = z1
    s2_ref[...] = jnp.dot(_bf(z1), _bf(w2_ref[...]),
                          preferred_element_type=jnp.float32)


def _gc2_body(adj_ref, s2_ref, z1_ref, clt_ref, enc_ref, q_ref):
    z2 = jnp.dot(_bf(adj_ref[...]), _bf(s2_ref[...]),
                 preferred_element_type=jnp.float32)
    enc = jnp.concatenate([z1_ref[...], z2], axis=1)
    enc_ref[...] = enc
    clt = clt_ref[...]
    en2 = jnp.sum(enc * enc, axis=1, keepdims=True)
    cn2 = jnp.sum(clt * clt, axis=0, keepdims=True)
    cross = jnp.dot(enc, clt, preferred_element_type=jnp.float32)
    dist = en2 - 2.0 * cross + cn2
    q = 1.0 / (1.0 + dist)
    q_ref[...] = q / jnp.sum(q, axis=1, keepdims=True)


def _dec_body(encb_ref, enc_ref, o_ref):
    s = lax.dot_general(_bf(encb_ref[...]), _bf(enc_ref[...]),
                        (((1,), (1,)), ((), ())),
                        preferred_element_type=jnp.float32)
    o_ref[...] = jax.nn.sigmoid(s)


@jax.jit
def kernel(x, adj, W1, W2, cluster_layer):
    bma = 512
    s1 = pl.pallas_call(
        _s1_body,
        grid=(N // bma,),
        in_specs=[
            pl.BlockSpec((bma, D), lambda i: (i, 0)),
            pl.BlockSpec((D, H1), lambda i: (0, 0)),
        ],
        out_specs=pl.BlockSpec((bma, H1), lambda i: (i, 0)),
        out_shape=jax.ShapeDtypeStruct((N, H1), jnp.float32),
    )(x, W1)

    bm = 256
    z1, s2 = pl.pallas_call(
        _gc1_body,
        grid=(N // bm,),
        in_specs=[
            pl.BlockSpec((bm, N), lambda i: (i, 0)),
            pl.BlockSpec((N, H1), lambda i: (0, 0)),
            pl.BlockSpec((H1, H2), lambda i: (0, 0)),
        ],
        out_specs=[
            pl.BlockSpec((bm, H1), lambda i: (i, 0)),
            pl.BlockSpec((bm, H2), lambda i: (i, 0)),
        ],
        out_shape=[
            jax.ShapeDtypeStruct((N, H1), jnp.float32),
            jax.ShapeDtypeStruct((N, H2), jnp.float32),
        ],
    )(adj, s1, W2)

    enc, q = pl.pallas_call(
        _gc2_body,
        grid=(N // bm,),
        in_specs=[
            pl.BlockSpec((bm, N), lambda i: (i, 0)),
            pl.BlockSpec((N, H2), lambda i: (0, 0)),
            pl.BlockSpec((bm, H1), lambda i: (i, 0)),
            pl.BlockSpec((HE, C), lambda i: (0, 0)),
        ],
        out_specs=[
            pl.BlockSpec((bm, HE), lambda i: (i, 0)),
            pl.BlockSpec((bm, C), lambda i: (i, 0)),
        ],
        out_shape=[
            jax.ShapeDtypeStruct((N, HE), jnp.float32),
            jax.ShapeDtypeStruct((N, C), jnp.float32),
        ],
    )(adj, s2, z1, cluster_layer.T)

    dec = pl.pallas_call(
        _dec_body,
        grid=(N // bm,),
        in_specs=[
            pl.BlockSpec((bm, HE), lambda i: (i, 0)),
            pl.BlockSpec((N, HE), lambda i: (0, 0)),
        ],
        out_specs=pl.BlockSpec((bm, N), lambda i: (i, 0)),
        out_shape=jax.ShapeDtypeStruct((N, N), jnp.float32),
    )(enc, enc)

    return (enc, dec, q)
